# baseline (speedup 1.0000x reference)
import functools
import math

import jax
import jax.numpy as jnp
from jax import lax
from jax.experimental import pallas as pl
from jax.experimental.pallas import tpu as pltpu
from jax.experimental.pallas import tpu_sc as plsc

BS = 2
N_CTX = 2048
N_HEADS = 12
WIDTH = 64
N_TOK = 128
HP = N_HEADS // 2
ROW = 2 * WIDTH

N_ROWS = BS * HP * N_TOK
NW = 32
RPW = N_ROWS // NW


def _sc_gather(qt, kt, vt, idx_all):
    mesh = plsc.VectorSubcoreMesh(core_axis_name="c", subcore_axis_name="s")
    out = jax.ShapeDtypeStruct((N_ROWS, ROW), jnp.float32)

    @functools.partial(
        pl.kernel,
        mesh=mesh,
        out_type=[out, out, out],
        scratch_types=[
            pltpu.VMEM((RPW,), jnp.int32),
            pltpu.VMEM((RPW, ROW), jnp.float32),
            pltpu.VMEM((RPW, ROW), jnp.float32),
            pltpu.VMEM((RPW, ROW), jnp.float32),
            pltpu.SemaphoreType.DMA,
        ],
    )
    def gather(qh, kh, vh, ih, qo, ko, vo, idx_v, qv, kv, vv, sem):
        wid = lax.axis_index("s") * 2 + lax.axis_index("c")
        base = wid * RPW
        pltpu.sync_copy(ih.at[pl.ds(base, RPW)], idx_v)
        cq = pltpu.async_copy(qh.at[idx_v], qv, sem)
        ck = pltpu.async_copy(kh.at[idx_v], kv, sem)
        cv = pltpu.async_copy(vh.at[idx_v], vv, sem)
        cq.wait()
        ck.wait()
        cv.wait()
        pltpu.sync_copy(qv, qo.at[pl.ds(base, RPW)])
        pltpu.sync_copy(kv, ko.at[pl.ds(base, RPW)])
        pltpu.sync_copy(vv, vo.at[pl.ds(base, RPW)])

    return gather(qt, kt, vt, idx_all)


def _attn_body(qr, kr, vr, outr):
    scale = 1.0 / math.sqrt(N_TOK)
    qb = qr[0, 0]
    kb = kr[0, 0]
    vb = vr[0, 0]
    for hp in range(2):
        sl = slice(hp * WIDTH, (hp + 1) * WIDTH)
        q2 = qb[:, sl]
        k2 = kb[:, sl]
        v2 = vb[:, sl]
        qT = q2.T
        d = jnp.abs(k2[:, :, None] - qT[None, :, :])
        a = -scale * jnp.sum(d, axis=1)
        a = a - jnp.max(a, axis=-1, keepdims=True)
        e = jnp.exp(a)
        s = jnp.sum(e, axis=-1, keepdims=True)
        p = e / s
        outr[0, :, 0, hp, :] = jnp.dot(p, v2, preferred_element_type=jnp.float32)


def _tc_attention(qg, kg, vg):
    spec = pl.BlockSpec((1, 1, N_TOK, ROW), lambda b, h: (b, h, 0, 0))
    out_spec = pl.BlockSpec((1, N_TOK, 1, 2, WIDTH), lambda b, h: (b, 0, h, 0, 0))
    return pl.pallas_call(
        _attn_body,
        grid=(BS, HP),
        in_specs=[spec, spec, spec],
        out_specs=out_spec,
        out_shape=jax.ShapeDtypeStruct((BS, N_TOK, HP, 2, WIDTH), jnp.float32),
    )(qg, kg, vg)


def kernel(q, k, v, indx):
    indx = indx.astype(jnp.int32)
    b_off = jnp.arange(BS, dtype=jnp.int32)[:, None, None] * N_CTX
    h_off = jnp.arange(HP, dtype=jnp.int32)[None, :, None]
    idx_all = ((indx[None, None, :] + b_off) * HP + h_off).reshape(-1)

    qt = q.reshape(BS * N_CTX * HP, ROW)
    kt = k.reshape(BS * N_CTX * HP, ROW)
    vt = v.reshape(BS * N_CTX * HP, ROW)

    qo, ko, vo = _sc_gather(qt, kt, vt, idx_all)
    qg = qo.reshape(BS, HP, N_TOK, ROW)
    kg = ko.reshape(BS, HP, N_TOK, ROW)
    vg = vo.reshape(BS, HP, N_TOK, ROW)

    out = _tc_attention(qg, kg, vg)
    return out.reshape(BS, N_TOK, N_HEADS, WIDTH)


# speedup vs baseline: 2.9226x; 2.9226x over previous
import math

import jax
import jax.numpy as jnp
from jax.experimental import pallas as pl
from jax.experimental.pallas import tpu as pltpu

BS = 2
N_CTX = 2048
N_HEADS = 12
WIDTH = 64
N_TOK = 128


def _split_hi_lo(x):
    hi = x.astype(jnp.bfloat16).astype(jnp.float32)
    return hi, x - hi


def _attn_body(ir, qr, kr, vr, outr, oh_ref):
    scale = 1.0 / math.sqrt(N_TOK)
    b = pl.program_id(0)
    h = pl.program_id(1)

    @pl.when(jnp.logical_and(b == 0, h == 0))
    def _():
        ids = jax.lax.broadcasted_iota(jnp.int32, (N_CTX, N_TOK), 0)
        oh_ref[...] = (ids == ir[...]).astype(jnp.float32)

    oh = oh_ref[...]
    qp = qr[0, 0]
    kp = kr[0, 0]
    vp = vr[0, 0]

    qs = jnp.dot(qp, oh, preferred_element_type=jnp.float32)
    ks = jnp.dot(kp, oh, preferred_element_type=jnp.float32)
    vhi, vlo = _split_hi_lo(vp)
    vs = (jnp.dot(vhi, oh, preferred_element_type=jnp.float32)
          + jnp.dot(vlo, oh, preferred_element_type=jnp.float32))

    kT = ks.T
    acc = jnp.zeros((N_TOK, N_TOK), jnp.float32)
    for w in range(WIDTH):
        acc = acc + jnp.abs(kT[:, w:w + 1] - qs[w:w + 1, :])
    a = -scale * acc
    a = a - jnp.max(a, axis=1, keepdims=True)
    e = jnp.exp(a)
    p = e / jnp.sum(e, axis=1, keepdims=True)

    vshi, vslo = _split_hi_lo(vs)
    dn = (((1,), (1,)), ((), ()))
    y = (jax.lax.dot_general(vshi, p, dn, preferred_element_type=jnp.float32)
         + jax.lax.dot_general(vslo, p, dn, preferred_element_type=jnp.float32))
    outr[0, 0] = y


def kernel(q, k, v, indx):
    qv = jnp.transpose(q, (0, 2, 3, 1))
    kv = jnp.transpose(k, (0, 2, 3, 1))
    vv = jnp.transpose(v, (0, 2, 3, 1))
    idx = indx.astype(jnp.int32).reshape(1, N_TOK)

    plane = pl.BlockSpec((1, 1, WIDTH, N_CTX), lambda b, h: (b, h, 0, 0))
    y = pl.pallas_call(
        _attn_body,
        grid=(BS, N_HEADS),
        in_specs=[
            pl.BlockSpec((1, N_TOK), lambda b, h: (0, 0)),
            plane, plane, plane,
        ],
        out_specs=pl.BlockSpec((1, 1, WIDTH, N_TOK), lambda b, h: (b, h, 0, 0)),
        out_shape=jax.ShapeDtypeStruct((BS, N_HEADS, WIDTH, N_TOK), jnp.float32),
        scratch_shapes=[pltpu.VMEM((N_CTX, N_TOK), jnp.float32)],
    )(idx, qv, kv, vv)
    return jnp.transpose(y, (0, 3, 1, 2))


# speedup vs baseline: 3.7747x; 1.2916x over previous
import math

import jax
import jax.numpy as jnp
from jax.experimental import pallas as pl
from jax.experimental.pallas import tpu as pltpu

BS = 2
N_CTX = 2048
N_HEADS = 12
WIDTH = 64
N_TOK = 128


def _split_hi_lo(x):
    hi = x.astype(jnp.bfloat16).astype(jnp.float32)
    return hi, x - hi


def _attn_body(ir, qr, kr, vr, outr, oh_ref):
    scale = 1.0 / math.sqrt(N_TOK)
    b = pl.program_id(0)
    h = pl.program_id(1)

    @pl.when(jnp.logical_and(b == 0, h == 0))
    def _():
        ids = jax.lax.broadcasted_iota(jnp.int32, (N_CTX, N_TOK), 0)
        oh_ref[...] = (ids == ir[...]).astype(jnp.float32)

    oh = oh_ref[...]
    qp = qr[0, 0]
    kp = kr[0, 0]
    vp = vr[0, 0]

    qs = jnp.dot(qp, oh, preferred_element_type=jnp.float32)
    ks = jnp.dot(kp, oh, preferred_element_type=jnp.float32)
    kT = ks.T
    qsb = qs.astype(jnp.bfloat16)
    kTb = kT.astype(jnp.bfloat16)

    accs = [jnp.zeros((8, N_TOK), jnp.float32) for _ in range(N_TOK // 8)]
    for w in range(WIDTH):
        qrow = qsb[w:w + 1, :]
        col = kTb[:, w:w + 1]
        for ci in range(N_TOK // 8):
            d = jnp.abs(col[8 * ci:8 * ci + 8, :] - qrow)
            accs[ci] = accs[ci] + d.astype(jnp.float32)
    a = -scale * jnp.concatenate(accs, axis=0)
    a = a - jnp.max(a, axis=1, keepdims=True)
    e = jnp.exp(a)
    p = e / jnp.sum(e, axis=1, keepdims=True)

    vhi, vlo = _split_hi_lo(vp)
    vs = (jnp.dot(vhi, oh, preferred_element_type=jnp.float32)
          + jnp.dot(vlo, oh, preferred_element_type=jnp.float32))

    vshi, vslo = _split_hi_lo(vs)
    dn = (((1,), (1,)), ((), ()))
    y = (jax.lax.dot_general(vshi, p, dn, preferred_element_type=jnp.float32)
         + jax.lax.dot_general(vslo, p, dn, preferred_element_type=jnp.float32))
    outr[0, 0] = y


def kernel(q, k, v, indx):
    qv = jnp.transpose(q, (0, 2, 3, 1))
    kv = jnp.transpose(k, (0, 2, 3, 1))
    vv = jnp.transpose(v, (0, 2, 3, 1))
    idx = indx.astype(jnp.int32).reshape(1, N_TOK)

    plane = pl.BlockSpec((1, 1, WIDTH, N_CTX), lambda b, h: (b, h, 0, 0))
    y = pl.pallas_call(
        _attn_body,
        grid=(BS, N_HEADS),
        in_specs=[
            pl.BlockSpec((1, N_TOK), lambda b, h: (0, 0)),
            plane, plane, plane,
        ],
        out_specs=pl.BlockSpec((1, 1, WIDTH, N_TOK), lambda b, h: (b, h, 0, 0)),
        out_shape=jax.ShapeDtypeStruct((BS, N_HEADS, WIDTH, N_TOK), jnp.float32),
        scratch_shapes=[pltpu.VMEM((N_CTX, N_TOK), jnp.float32)],
    )(idx, qv, kv, vv)
    return jnp.transpose(y, (0, 3, 1, 2))


# speedup vs baseline: 4.1524x; 1.1001x over previous
import math

import jax
import jax.numpy as jnp
from jax.experimental import pallas as pl
from jax.experimental.pallas import tpu as pltpu

BS = 2
N_CTX = 2048
N_HEADS = 12
WIDTH = 64
N_TOK = 128
HPG = 2


def _split_hi_lo(x):
    hi = x.astype(jnp.bfloat16)
    lo = (x - hi.astype(jnp.float32)).astype(jnp.bfloat16)
    return hi, lo


def _one_head(oh, qp, kp, vp):
    scale = 1.0 / math.sqrt(N_TOK)
    qs = jnp.dot(qp.astype(jnp.bfloat16), oh, preferred_element_type=jnp.float32)
    ks = jnp.dot(kp.astype(jnp.bfloat16), oh, preferred_element_type=jnp.float32)
    kT = ks.T
    qsb = qs.astype(jnp.bfloat16)
    kTb = kT.astype(jnp.bfloat16)

    accs = [jnp.zeros((8, N_TOK), jnp.float32) for _ in range(N_TOK // 8)]
    for w in range(WIDTH):
        qrow = qsb[w:w + 1, :]
        col = kTb[:, w:w + 1]
        for ci in range(N_TOK // 8):
            d = jnp.abs(col[8 * ci:8 * ci + 8, :] - qrow)
            accs[ci] = accs[ci] + d.astype(jnp.float32)
    a = -scale * jnp.concatenate(accs, axis=0)
    a = a - jnp.max(a, axis=1, keepdims=True)
    e = jnp.exp(a)
    p = e / jnp.sum(e, axis=1, keepdims=True)

    vhi, vlo = _split_hi_lo(vp)
    vs = (jnp.dot(vhi, oh, preferred_element_type=jnp.float32)
          + jnp.dot(vlo, oh, preferred_element_type=jnp.float32))

    vshi, vslo = _split_hi_lo(vs)
    pb = p.astype(jnp.bfloat16)
    dn = (((1,), (1,)), ((), ()))
    return (jax.lax.dot_general(vshi, pb, dn, preferred_element_type=jnp.float32)
            + jax.lax.dot_general(vslo, pb, dn, preferred_element_type=jnp.float32))


def _attn_body(ir, qr, kr, vr, outr, oh_ref):
    b = pl.program_id(0)
    h = pl.program_id(1)

    @pl.when(jnp.logical_and(b == 0, h == 0))
    def _():
        ids = jax.lax.broadcasted_iota(jnp.int32, (N_CTX, N_TOK), 0)
        oh_ref[...] = (ids == ir[...]).astype(jnp.bfloat16)

    oh = oh_ref[...]
    for hh in range(HPG):
        outr[0, hh] = _one_head(oh, qr[0, hh], kr[0, hh], vr[0, hh])


def kernel(q, k, v, indx):
    qv = jnp.transpose(q, (0, 2, 3, 1))
    kv = jnp.transpose(k, (0, 2, 3, 1))
    vv = jnp.transpose(v, (0, 2, 3, 1))
    idx = indx.astype(jnp.int32).reshape(1, N_TOK)

    plane = pl.BlockSpec((1, HPG, WIDTH, N_CTX), lambda b, h: (b, h, 0, 0))
    y = pl.pallas_call(
        _attn_body,
        grid=(BS, N_HEADS // HPG),
        in_specs=[
            pl.BlockSpec((1, N_TOK), lambda b, h: (0, 0)),
            plane, plane, plane,
        ],
        out_specs=pl.BlockSpec((1, HPG, WIDTH, N_TOK), lambda b, h: (b, h, 0, 0)),
        out_shape=jax.ShapeDtypeStruct((BS, N_HEADS, WIDTH, N_TOK), jnp.float32),
        scratch_shapes=[pltpu.VMEM((N_CTX, N_TOK), jnp.bfloat16)],
    )(idx, qv, kv, vv)
    return jnp.transpose(y, (0, 3, 1, 2))
